# baseline (speedup 1.0000x reference)
import jax
import jax.numpy as jnp
from jax import lax
from jax.experimental import pallas as pl
from jax.experimental.pallas import tpu as pltpu
from jax.experimental.pallas import tpu_sc as plsc

K = 8
B = 4.0
ADJ = 0.01
MIN_DERIVATIVE = 0.001
PAD_LO = -40000.0
PAD_HI = 40000.0

NC = 2
NS = 16
L = 16
NW = NC * NS

CH = 2048
GRP = CH // L
TBL = 24 * L

A_SC = 2.0 * B / (1.0 + ADJ)
BETA = 2.0 * B * (ADJ / K) / (1.0 + ADJ)


def _softplus2(t1, t2):
    e1 = jnp.exp(-jnp.abs(t1))
    e2 = jnp.exp(-jnp.abs(t2))
    a = e1 + 2.0
    b = e2 + 2.0
    r = 1.0 / (a * b)
    outs = []
    for t, e, other in ((t1, e1, b), (t2, e2, a)):
        z = e * other * r
        z2 = z * z
        p = 2.0 / 7.0 + z2 * (2.0 / 9.0)
        p = 2.0 / 5.0 + z2 * p
        p = 2.0 / 3.0 + z2 * p
        p = 2.0 + z2 * p
        outs.append(jnp.maximum(t, 0.0) + z * p)
    return outs


def _prefix8(w):
    s01 = w[0] + w[1]
    s23 = w[2] + w[3]
    s45 = w[4] + w[5]
    s67 = w[6] + w[7]
    s03 = s01 + s23
    s47 = s45 + s67
    total = s03 + s47
    c = [w[0], s01, s01 + w[2], s03, s03 + w[4], s03 + s45,
         s03 + s45 + w[6], total]
    return c


def _sc_body(x_hbm, p_hbm, y_hbm, xv0, pv0, yv0, xv1, pv1, yv1,
             sin0, sin1, sout0, sout1):
    wid = lax.axis_index("s") * NC + lax.axis_index("c")
    per_w = x_hbm.shape[0] // NW
    nch = per_w // CH
    bufs = ((xv0, pv0, yv0, sin0, sout0), (xv1, pv1, yv1, sin1, sout1))

    def group(g, xv, pv, yv):
        eoff = g * L
        x = xv[pl.ds(eoff, L)]

        def expsum(off):
            w = [jnp.exp(pv[off + j, pl.ds(eoff, L)]) for j in range(K)]
            h = 0.5 * w[0]
            c = _prefix8([h] + w[1:])
            return h, c

        hx, cx = expsum(0)
        hy, cy = expsum(K)
        scale_x = A_SC / (cx[-1] + hx)
        scale_y = A_SC / (cy[-1] + hy)

        def positions(c, scale):
            return [c[j] * scale + (BETA * (j + 0.5) - B) for j in range(K)]

        xp = positions(cx, scale_x)
        yp = positions(cy, scale_y)

        m = [xp[j] < x for j in range(K)]
        mlo = x > -B
        mhi = x > B

        def sel_lo(p):
            v = jnp.where(mlo, -B, PAD_LO)
            for j in range(K):
                v = jnp.where(m[j], p[j], v)
            return jnp.where(mhi, B, v)

        def sel_hi(p):
            v = jnp.where(mhi, PAD_HI, B)
            for j in reversed(range(K)):
                v = jnp.where(m[j], v, p[j])
            return jnp.where(mlo, v, -B)

        xk = sel_lo(xp)
        xk1 = sel_hi(xp)
        yk = sel_lo(yp)
        yk1 = sel_hi(yp)

        rows = [pv[2 * K + i, pl.ds(eoff, L)] for i in range(K - 1)]
        pdk = rows[0]
        pdk1 = rows[0]
        for i in range(1, K - 1):
            pdk = jnp.where(m[i], rows[i], pdk)
            pdk1 = jnp.where(m[i - 1], rows[i], pdk1)
        interior = jnp.logical_and(mlo, jnp.logical_and(
            m[0], jnp.logical_not(m[K - 1])))
        interior1 = jnp.logical_and(mlo, jnp.logical_not(m[K - 2]))
        sp, sp1 = _softplus2(pdk, pdk1)
        dk = jnp.where(interior, sp + MIN_DERIVATIVE, 1.0)
        dk1 = jnp.where(interior1, sp1 + MIN_DERIVATIVE, 1.0)

        dx = xk1 - xk
        dy = yk1 - yk
        t = x - xk
        u = t * (dx - t)
        num = dy * (dy * (t * t) + (dk * dx) * u)
        den = dy * (dx * dx) + ((dk + dk1) * dx - 2.0 * dy) * u
        yv[pl.ds(g * L, L)] = yk + num / den

    base_w = wid * per_w

    def start_in(ci, b):
        xv, pv, _, sin, _ = bufs[b]
        base = base_w + ci * CH
        pltpu.async_copy(x_hbm.at[pl.ds(base, CH)], xv, sin)
        pltpu.async_copy(p_hbm.at[:, pl.ds(base, CH)], pv, sin)

    def wait_in(b):
        xv, pv, _, sin, _ = bufs[b]
        pltpu.make_async_copy(x_hbm.at[pl.ds(0, CH)], xv, sin).wait()
        pltpu.make_async_copy(p_hbm.at[:, pl.ds(0, CH)], pv, sin).wait()

    def start_out(ci, b):
        _, _, yv, _, sout = bufs[b]
        pltpu.async_copy(yv, y_hbm.at[pl.ds(base_w + ci * CH, CH)], sout)

    def wait_out(b):
        _, _, yv, _, sout = bufs[b]
        pltpu.make_async_copy(yv, y_hbm.at[pl.ds(0, CH)], sout).wait()

    def compute(xv, pv, yv):
        @plsc.parallel_loop(0, GRP, 1, unroll=2)
        def _(g):
            group(g, xv, pv, yv)

    start_in(0, 0)
    start_in(1, 1)

    def chunk_pair(i, carry):
        for b in range(2):
            ci = 2 * i + b
            xv, pv, yv, _, _ = bufs[b]
            wait_in(b)

            @pl.when(i > 0)
            def _():
                wait_out(b)

            compute(xv, pv, yv)
            start_out(ci, b)

            @pl.when(i < (nch // 2 - 1))
            def _():
                start_in(ci + 2, b)

        return carry

    lax.fori_loop(0, nch // 2, chunk_pair, 0)
    wait_out(0)
    wait_out(1)


@jax.jit
def kernel(x, params):
    f = pl.kernel(
        _sc_body,
        out_type=jax.ShapeDtypeStruct(x.shape, jnp.float32),
        mesh=plsc.VectorSubcoreMesh(core_axis_name="c", subcore_axis_name="s"),
        compiler_params=pltpu.CompilerParams(needs_layout_passes=False),
        scratch_types=[
            pltpu.VMEM((CH,), jnp.float32),
            pltpu.VMEM((23, CH), jnp.float32),
            pltpu.VMEM((CH,), jnp.float32),
            pltpu.VMEM((CH,), jnp.float32),
            pltpu.VMEM((23, CH), jnp.float32),
            pltpu.VMEM((CH,), jnp.float32),
            pltpu.SemaphoreType.DMA,
            pltpu.SemaphoreType.DMA,
            pltpu.SemaphoreType.DMA,
            pltpu.SemaphoreType.DMA,
        ],
    )
    return f(x, params.T)


# speedup vs baseline: 1.0536x; 1.0536x over previous
import jax
import jax.numpy as jnp
from jax import lax
from jax.experimental import pallas as pl
from jax.experimental.pallas import tpu as pltpu
from jax.experimental.pallas import tpu_sc as plsc

K = 8
B = 4.0
ADJ = 0.01
MIN_DERIVATIVE = 0.001
PAD_LO = -40000.0
PAD_HI = 40000.0

NC = 2
NS = 16
L = 16
NW = NC * NS

CH = 1024
GRP = CH // L
TBL = 24 * L

A_SC = 2.0 * B / (1.0 + ADJ)
BETA = 2.0 * B * (ADJ / K) / (1.0 + ADJ)


def _softplus2(t1, t2):
    e1 = jnp.exp(-jnp.abs(t1))
    e2 = jnp.exp(-jnp.abs(t2))
    a = e1 + 2.0
    b = e2 + 2.0
    r = 1.0 / (a * b)
    outs = []
    for t, e, other in ((t1, e1, b), (t2, e2, a)):
        z = e * other * r
        z2 = z * z
        p = 2.0 / 7.0 + z2 * (2.0 / 9.0)
        p = 2.0 / 5.0 + z2 * p
        p = 2.0 / 3.0 + z2 * p
        p = 2.0 + z2 * p
        outs.append(jnp.maximum(t, 0.0) + z * p)
    return outs


def _prefix8(w):
    s01 = w[0] + w[1]
    s23 = w[2] + w[3]
    s45 = w[4] + w[5]
    s67 = w[6] + w[7]
    s03 = s01 + s23
    s47 = s45 + s67
    total = s03 + s47
    c = [w[0], s01, s01 + w[2], s03, s03 + w[4], s03 + s45,
         s03 + s45 + w[6], total]
    return c


def _sc_body(x_hbm, p_hbm, y_hbm, xv0, pv0, yv0, xv1, pv1, yv1,
             sin0, sin1, sout0, sout1):
    wid = lax.axis_index("s") * NC + lax.axis_index("c")
    per_w = x_hbm.shape[0] // NW
    nch = per_w // CH
    bufs = ((xv0, pv0, yv0, sin0, sout0), (xv1, pv1, yv1, sin1, sout1))

    def group(g, xv, pv, yv):
        eoff = g * L
        x = xv[pl.ds(eoff, L)]

        def expsum(off):
            w = [jnp.exp(pv[off + j, pl.ds(eoff, L)]) for j in range(K)]
            h = 0.5 * w[0]
            c = _prefix8([h] + w[1:])
            return h, c

        hx, cx = expsum(0)
        hy, cy = expsum(K)
        sx = cx[-1] + hx
        sy = cy[-1] + hy
        r = 1.0 / (sx * sy)
        scale_x = (A_SC * r) * sy
        scale_y = (A_SC * r) * sx

        def positions(c, scale):
            return [c[j] * scale + (BETA * (j + 0.5) - B) for j in range(K)]

        xp = positions(cx, scale_x)
        yp = positions(cy, scale_y)

        m = [xp[j] < x for j in range(K)]
        mlo = x > -B
        mhi = x > B

        def sel_lo(p):
            v = jnp.where(mlo, -B, PAD_LO)
            for j in range(K):
                v = jnp.where(m[j], p[j], v)
            return jnp.where(mhi, B, v)

        def sel_hi(p):
            v = jnp.where(mhi, PAD_HI, B)
            for j in reversed(range(K)):
                v = jnp.where(m[j], v, p[j])
            return jnp.where(mlo, v, -B)

        xk = sel_lo(xp)
        xk1 = sel_hi(xp)
        yk = sel_lo(yp)
        yk1 = sel_hi(yp)

        rows = [pv[2 * K + i, pl.ds(eoff, L)] for i in range(K - 1)]
        pdk = rows[0]
        pdk1 = rows[0]
        for i in range(1, K - 1):
            pdk = jnp.where(m[i], rows[i], pdk)
            pdk1 = jnp.where(m[i - 1], rows[i], pdk1)
        interior = jnp.logical_and(mlo, jnp.logical_and(
            m[0], jnp.logical_not(m[K - 1])))
        interior1 = jnp.logical_and(mlo, jnp.logical_not(m[K - 2]))
        sp, sp1 = _softplus2(pdk, pdk1)
        dk = jnp.where(interior, sp + MIN_DERIVATIVE, 1.0)
        dk1 = jnp.where(interior1, sp1 + MIN_DERIVATIVE, 1.0)

        dx = xk1 - xk
        dy = yk1 - yk
        t = x - xk
        u = t * (dx - t)
        num = dy * (dy * (t * t) + (dk * dx) * u)
        den = dy * (dx * dx) + ((dk + dk1) * dx - 2.0 * dy) * u
        yv[pl.ds(g * L, L)] = yk + num / den

    base_w = wid * per_w

    def start_in(ci, b):
        xv, pv, _, sin, _ = bufs[b]
        base = base_w + ci * CH
        pltpu.async_copy(x_hbm.at[pl.ds(base, CH)], xv, sin)
        pltpu.async_copy(p_hbm.at[:, pl.ds(base, CH)], pv, sin)

    def wait_in(b):
        xv, pv, _, sin, _ = bufs[b]
        pltpu.make_async_copy(x_hbm.at[pl.ds(0, CH)], xv, sin).wait()
        pltpu.make_async_copy(p_hbm.at[:, pl.ds(0, CH)], pv, sin).wait()

    def start_out(ci, b):
        _, _, yv, _, sout = bufs[b]
        pltpu.async_copy(yv, y_hbm.at[pl.ds(base_w + ci * CH, CH)], sout)

    def wait_out(b):
        _, _, yv, _, sout = bufs[b]
        pltpu.make_async_copy(yv, y_hbm.at[pl.ds(0, CH)], sout).wait()

    def compute(xv, pv, yv):
        @plsc.parallel_loop(0, GRP, 1, unroll=2)
        def _(g):
            group(g, xv, pv, yv)

    start_in(0, 0)
    start_in(1, 1)

    def chunk_pair(i, carry):
        for b in range(2):
            ci = 2 * i + b
            xv, pv, yv, _, _ = bufs[b]
            wait_in(b)

            @pl.when(i > 0)
            def _():
                wait_out(b)

            compute(xv, pv, yv)
            start_out(ci, b)

            @pl.when(i < (nch // 2 - 1))
            def _():
                start_in(ci + 2, b)

        return carry

    lax.fori_loop(0, nch // 2, chunk_pair, 0)
    wait_out(0)
    wait_out(1)


@jax.jit
def kernel(x, params):
    f = pl.kernel(
        _sc_body,
        out_type=jax.ShapeDtypeStruct(x.shape, jnp.float32),
        mesh=plsc.VectorSubcoreMesh(core_axis_name="c", subcore_axis_name="s"),
        compiler_params=pltpu.CompilerParams(needs_layout_passes=False),
        scratch_types=[
            pltpu.VMEM((CH,), jnp.float32),
            pltpu.VMEM((23, CH), jnp.float32),
            pltpu.VMEM((CH,), jnp.float32),
            pltpu.VMEM((CH,), jnp.float32),
            pltpu.VMEM((23, CH), jnp.float32),
            pltpu.VMEM((CH,), jnp.float32),
            pltpu.SemaphoreType.DMA,
            pltpu.SemaphoreType.DMA,
            pltpu.SemaphoreType.DMA,
            pltpu.SemaphoreType.DMA,
        ],
    )
    return f(x, params.T)
